# baseline (speedup 1.0000x reference)
import jax
import jax.numpy as jnp
from jax.experimental import pallas as pl
from jax.experimental.pallas import tpu as pltpu

_H = 1000
_C = 128
_NSEM = 8


def _copy_rows_body(table_ref, *rest):
    outs = rest[:_H]
    vmem = rest[_H]
    sem_in = rest[_H + 1]
    sems = rest[_H + 2:_H + 2 + _NSEM]
    pltpu.make_async_copy(table_ref, vmem, sem_in).start()
    pltpu.make_async_copy(table_ref, vmem, sem_in).wait()
    copies = [
        pltpu.make_async_copy(
            vmem.at[pl.ds(i, 1)] if i % 5 < 3 else table_ref.at[pl.ds(i, 1)],
            outs[i],
            sems[i % _NSEM],
        )
        for i in range(_H)
    ]
    for c in copies:
        c.start()
    for c in copies:
        c.wait()


def kernel(table):
    outs = pl.pallas_call(
        _copy_rows_body,
        in_specs=[pl.BlockSpec(memory_space=pl.ANY)],
        out_specs=[pl.BlockSpec(memory_space=pl.ANY)] * _H,
        out_shape=[jax.ShapeDtypeStruct((1, _C), jnp.float32)] * _H,
        scratch_shapes=[pltpu.VMEM((_H, _C), jnp.float32)]
        + [pltpu.SemaphoreType.DMA] * (1 + _NSEM),
    )(table)
    return {f"embeds_bb_{i}": {"codes": outs[i]} for i in range(_H)}


# speedup vs baseline: 1.1660x; 1.1660x over previous
import jax
import jax.numpy as jnp
from jax.experimental import pallas as pl
from jax.experimental.pallas import tpu as pltpu

_H = 1000
_C = 128
_NSEM = 32


def _copy_rows_body(table_ref, *rest):
    outs = rest[:_H]
    vmem = rest[_H]
    sem_in = rest[_H + 1]
    sems = rest[_H + 2:_H + 2 + _NSEM]
    pltpu.make_async_copy(table_ref, vmem, sem_in).start()
    pltpu.make_async_copy(table_ref, vmem, sem_in).wait()
    copies = [
        pltpu.make_async_copy(vmem.at[pl.ds(i, 1)], outs[i], sems[i % _NSEM])
        for i in range(_H)
    ]
    for c in copies:
        c.start()
    for c in copies:
        c.wait()


def kernel(table):
    outs = pl.pallas_call(
        _copy_rows_body,
        in_specs=[pl.BlockSpec(memory_space=pl.ANY)],
        out_specs=[pl.BlockSpec(memory_space=pl.ANY)] * _H,
        out_shape=[jax.ShapeDtypeStruct((1, _C), jnp.float32)] * _H,
        scratch_shapes=[pltpu.VMEM((_H, _C), jnp.float32)]
        + [pltpu.SemaphoreType.DMA] * (1 + _NSEM),
    )(table)
    return {f"embeds_bb_{i}": {"codes": outs[i]} for i in range(_H)}


# speedup vs baseline: 1.1996x; 1.0288x over previous
import jax
import jax.numpy as jnp
from jax.experimental import pallas as pl
from jax.experimental.pallas import tpu as pltpu

_H = 1000
_C = 128
_NSEM = 32
_NCHUNK = 4
_CHUNK = 250


def _copy_rows_body(table_ref, *rest):
    outs = rest[:_H]
    vmem = rest[_H]
    stage_sems = rest[_H + 1:_H + 1 + _NCHUNK]
    sems = rest[_H + 1 + _NCHUNK:_H + 1 + _NCHUNK + _NSEM]
    stages = [
        pltpu.make_async_copy(
            table_ref.at[pl.ds(k * _CHUNK, _CHUNK)],
            vmem.at[pl.ds(k * _CHUNK, _CHUNK)],
            stage_sems[k],
        )
        for k in range(_NCHUNK)
    ]
    for s in stages:
        s.start()
    copies = [
        pltpu.make_async_copy(vmem.at[pl.ds(i, 1)], outs[i], sems[i % _NSEM])
        for i in range(_H)
    ]
    for k in range(_NCHUNK):
        stages[k].wait()
        for c in copies[k * _CHUNK:(k + 1) * _CHUNK]:
            c.start()
    for c in copies:
        c.wait()


def kernel(table):
    outs = pl.pallas_call(
        _copy_rows_body,
        in_specs=[pl.BlockSpec(memory_space=pl.ANY)],
        out_specs=[pl.BlockSpec(memory_space=pl.ANY)] * _H,
        out_shape=[jax.ShapeDtypeStruct((1, _C), jnp.float32)] * _H,
        scratch_shapes=[pltpu.VMEM((_H, _C), jnp.float32)]
        + [pltpu.SemaphoreType.DMA] * (_NCHUNK + _NSEM),
    )(table)
    return {f"embeds_bb_{i}": {"codes": outs[i]} for i in range(_H)}
